# baseline (speedup 1.0000x reference)
import functools

import jax
import jax.numpy as jnp
from jax import lax
from jax.experimental import pallas as pl
from jax.experimental.pallas import tpu as pltpu
from jax.experimental.pallas import tpu_sc as plsc

_L = 16


@functools.lru_cache(maxsize=None)
def _make_lookup(BF, F, V, D, NC, NS):
    NW = NC * NS
    per_w = BF // NW
    chunk = per_w
    nchunk = 1
    while chunk * D > 64 * 1024 or chunk % F != 0:
        nchunk *= 2
        chunk = per_w // nchunk
    assert chunk % F == 0 and chunk % 8 == 0 and chunk * nchunk == per_w

    mesh = plsc.VectorSubcoreMesh(core_axis_name="c", subcore_axis_name="s")

    @functools.partial(
        pl.kernel,
        mesh=mesh,
        out_type=jax.ShapeDtypeStruct((BF, D), jnp.float32),
        compiler_params=pltpu.CompilerParams(use_tc_tiling_on_sc=False),
        scratch_types=[
            pltpu.VMEM((chunk,), jnp.int32),
            pltpu.VMEM((chunk,), jnp.int32),
            pltpu.VMEM((chunk, D), jnp.float32),
            pltpu.SemaphoreType.DMA,
        ],
    )
    def lookup(idx_hbm, tab_hbm, foff_hbm, out_hbm, foff_v, gidx_v, rows_v, sem):
        wid = lax.axis_index("s") * NC + lax.axis_index("c")
        base = wid * per_w
        pltpu.sync_copy(foff_hbm, foff_v)
        for c in range(nchunk):
            cbase = base + c * chunk
            pltpu.sync_copy(idx_hbm.at[pl.ds(cbase, chunk)], gidx_v)

            def step(i, carry):
                s = pl.ds(i * _L, _L)
                gidx_v[s] = jnp.minimum(gidx_v[s], V - 1) + foff_v[s]
                return carry

            lax.fori_loop(0, chunk // _L, step, 0)
            pltpu.async_copy(tab_hbm.at[gidx_v], rows_v, sem).wait()
            pltpu.sync_copy(rows_v, out_hbm.at[pl.ds(cbase, chunk)])

    return lookup, chunk


def kernel(tab_data, tables, batch_size):
    F, V, D = tables.shape
    B = tab_data.shape[0]
    BF = B * F
    info = plsc.get_sparse_core_info()
    NC, NS = info.num_cores, info.num_subcores

    lookup, chunk = _make_lookup(BF, F, V, D, NC, NS)
    idx_flat = tab_data.reshape(BF).astype(jnp.int32)
    tab_flat = tables.reshape(F * V, D)
    foff = (jnp.arange(chunk, dtype=jnp.int32) % F) * V
    out_flat = lookup(idx_flat, tab_flat, foff)
    return out_flat.reshape(B, F, D)


# speedup vs baseline: 1.1877x; 1.1877x over previous
import functools

import jax
import jax.numpy as jnp
from jax import lax
from jax.experimental import pallas as pl
from jax.experimental.pallas import tpu as pltpu
from jax.experimental.pallas import tpu_sc as plsc

_L = 16


@functools.lru_cache(maxsize=None)
def _make_lookup(B, F, V, D, NC, NS):
    NW = NC * NS
    BW = B // NW
    WB = 256
    NWIN = BW // WB
    SR = (F * V) // 8
    assert BW % WB == 0 and WB % _L == 0 and D == _L

    mesh = plsc.VectorSubcoreMesh(core_axis_name="c", subcore_axis_name="s")

    @functools.partial(
        pl.kernel,
        mesh=mesh,
        out_type=jax.ShapeDtypeStruct((F * D, B), jnp.float32),
        compiler_params=pltpu.CompilerParams(
            use_tc_tiling_on_sc=True, needs_layout_passes=False
        ),
        scratch_types=[
            pltpu.VMEM((WB,), jnp.int32),
            pltpu.VMEM((WB,), jnp.int32),
            pltpu.VMEM((WB, 128), jnp.float32),
            pltpu.VMEM((D, WB), jnp.float32),
            pltpu.SemaphoreType.DMA,
        ],
    )
    def lookup(idx_hbm, tab_hbm, out_hbm, sr_v, low_v, rows_v, stg_v, sem):
        wid = lax.axis_index("s") * NC + lax.axis_index("c")
        b0w = wid * BW
        iota = lax.iota(jnp.int32, _L)

        def unit(u, carry):
            f = u // NWIN
            b0 = b0w + (u % NWIN) * WB
            pltpu.sync_copy(idx_hbm.at[pl.ds(f * B + b0, WB)], sr_v)
            for i in range(WB // _L):
                s = pl.ds(i * _L, _L)
                fv = jnp.minimum(sr_v[s], V - 1) + f * V
                sr_v[s] = fv >> 3
                low_v[s] = (fv & 7) * _L
            pltpu.async_copy(tab_hbm.at[sr_v], rows_v, sem).wait()
            for k0 in range(0, WB, _L):
                rowv = k0 + iota
                colv = low_v[pl.ds(k0, _L)]
                for d in range(D):
                    vals = plsc.load_gather(rows_v, [rowv, colv + d])
                    stg_v[d, pl.ds(k0, _L)] = vals
            pltpu.sync_copy(stg_v, out_hbm.at[pl.ds(f * D, D), pl.ds(b0, WB)])
            return carry

        lax.fori_loop(0, F * NWIN, unit, 0)

    return lookup


def kernel(tab_data, tables, batch_size):
    F, V, D = tables.shape
    B = tab_data.shape[0]
    info = plsc.get_sparse_core_info()
    NC, NS = info.num_cores, info.num_subcores

    lookup = _make_lookup(B, F, V, D, NC, NS)
    idxf = tab_data.astype(jnp.int32).T.reshape(F * B)
    tab8 = tables.reshape(F * V // 8, 8 * D)
    out = lookup(idxf, tab8)
    return out.reshape(F, D, B).transpose(2, 0, 1)
